# Initial kernel scaffold; baseline (speedup 1.0000x reference)
import jax
import jax.numpy as jnp
from jax.experimental import pallas as pl


def kernel(x, rtg_w, rtg_b, obs_w, obs_b, act_w, act_b, fc_w, fc_b):
    raise NotImplementedError("write your pallas kernel here")


# speedup vs baseline: 1.5916x; 1.5916x over previous
import functools

import jax
import jax.numpy as jnp
from jax.experimental import pallas as pl
from jax.experimental.pallas import tpu as pltpu


def _fused_kernel(x_ref, tmod_ref, w_ref, be_ref, fcw_ref, out_ref, *, W, TB):
    tmod = tmod_ref[...]
    is1 = tmod == 1
    is2 = tmod == 2

    def sel(v):
        return jnp.where(is2, v[2], jnp.where(is1, v[1], v[0]))

    x = x_ref[...]
    T = x.shape[1]
    a = x * sel(w_ref[:, W - 1])[None]
    for k in range(W - 1):
        d = W - 1 - k
        wk = sel(w_ref[:, k])
        contrib = x[:, : T - d, :] * wk[None, d:, :]
        a = a + jnp.pad(contrib, ((0, 0), (d, 0), (0, 0)))

    C = x.shape[2]
    y = jnp.dot(a.reshape(TB * T, C), fcw_ref[...],
                preferred_element_type=jnp.float32)
    out_ref[...] = (y.reshape(TB, T, C) + sel(be_ref[...])[None]).astype(
        out_ref.dtype)


def kernel(x, rtg_w, rtg_b, obs_w, obs_b, act_w, act_b, fc_w, fc_b):
    B, T, C = x.shape
    W = rtg_w.shape[1]

    batch_blocks = 8 if B % 8 == 0 else (2 if B % 2 == 0 else 1)
    TB = B // batch_blocks

    w_stack = jnp.transpose(jnp.stack([rtg_w, obs_w, act_w]), (0, 2, 1))
    fcw_t = fc_w.T
    bias_eff = jnp.stack([rtg_b, obs_b, act_b]) @ fcw_t + fc_b[None, :]
    tmod3 = (jnp.arange(T, dtype=jnp.int32) % 3).reshape(T, 1)

    out = pl.pallas_call(
        functools.partial(_fused_kernel, W=W, TB=TB),
        out_shape=jax.ShapeDtypeStruct((B, T, C), x.dtype),
        grid=(batch_blocks,),
        in_specs=[
            pl.BlockSpec((TB, T, C), lambda i: (i, 0, 0)),
            pl.BlockSpec((T, 1), lambda i: (0, 0)),
            pl.BlockSpec((3, W, C), lambda i: (0, 0, 0)),
            pl.BlockSpec((3, C), lambda i: (0, 0)),
            pl.BlockSpec((C, C), lambda i: (0, 0)),
        ],
        out_specs=pl.BlockSpec((TB, T, C), lambda i: (i, 0, 0)),
        compiler_params=pltpu.CompilerParams(
            dimension_semantics=("parallel",)),
    )(x, tmod3, w_stack, bias_eff, fcw_t)
    return out
